# Initial kernel scaffold; baseline (speedup 1.0000x reference)
import jax
import jax.numpy as jnp
from jax.experimental import pallas as pl


def kernel(feat, props):
    raise NotImplementedError("write your pallas kernel here")


# speedup vs baseline: 5.0924x; 5.0924x over previous
import functools

import jax
import jax.numpy as jnp
from jax.experimental import pallas as pl

_C, _H, _W = 96, 64, 64
_OUT = 7
_SCALE = 1.0 / 16.0
_N = 1000
_B = 8


def _interp_matrix(c1, c2, dim):
    b = c1.shape[0]
    roi = jnp.maximum(c2 - c1, 1.0)
    binw = roi / _OUT
    k = jax.lax.broadcasted_iota(jnp.int32, (1, 14), 1).astype(jnp.float32)
    pos = c1 + (k + 0.5) * 0.5 * binw
    valid = (pos > -1.0) & (pos < dim)
    x = jnp.maximum(pos, 0.0)
    xl0 = jnp.floor(x).astype(jnp.int32)
    cond = xl0 >= dim - 1
    xl = jnp.where(cond, dim - 1, xl0)
    xh = jnp.where(cond, dim - 1, xl0 + 1)
    xv = jnp.where(cond, xl.astype(jnp.float32), x)
    lx = xv - xl.astype(jnp.float32)
    hx = 1.0 - lx
    vf = valid.astype(jnp.float32)
    hx = hx * vf
    lx = lx * vf
    cols = jax.lax.broadcasted_iota(jnp.int32, (b, 14, dim), 2)
    r = (jnp.where(cols == xl[:, :, None], hx[:, :, None], 0.0)
         + jnp.where(cols == xh[:, :, None], lx[:, :, None], 0.0))
    return 0.5 * r.reshape(b, _OUT, 2, dim).sum(axis=2)


def _roi_kernel(props_ref, f2_ref, out_ref):
    p = props_ref[...]
    x1 = p[:, 0:1] * _SCALE
    y1 = p[:, 1:2] * _SCALE
    x2 = p[:, 2:3] * _SCALE
    y2 = p[:, 3:4] * _SCALE
    ay = _interp_matrix(y1, y2, _H)
    ax = _interp_matrix(x1, x2, _W)

    f3 = f2_ref[...]
    t = jax.lax.dot_general(
        ay.reshape(_B * _OUT, _H), f3,
        (((1,), (1,)), ((), ())),
        preferred_element_type=jnp.float32)
    t = t.reshape(_B, _OUT, _C, _W)
    o = jax.lax.dot_general(
        t, ax,
        (((3,), (2,)), ((0,), (0,))),
        preferred_element_type=jnp.float32)
    out_ref[...] = o.transpose(0, 2, 1, 3)


@jax.jit
def kernel(feat, props):
    propst = props.T
    grid = _N // _B
    out = pl.pallas_call(
        _roi_kernel,
        grid=(grid,),
        in_specs=[
            pl.BlockSpec((_B, 4), lambda i: (i, 0)),
            pl.BlockSpec((_C, _H, _W), lambda i: (0, 0, 0)),
        ],
        out_specs=pl.BlockSpec((_B, _C, _OUT, _OUT), lambda i: (i, 0, 0, 0)),
        out_shape=jax.ShapeDtypeStruct((_N, _C, _OUT, _OUT), jnp.float32),
    )(propst, feat[0])
    return out


# speedup vs baseline: 6.1403x; 1.2058x over previous
import functools

import jax
import jax.numpy as jnp
from jax.experimental import pallas as pl

_C, _H, _W = 96, 64, 64
_OUT = 7
_SCALE = 1.0 / 16.0
_N = 1000
_B = 8


def _interp_matrix(c1, c2, dim):
    b = c1.shape[0]
    roi = jnp.maximum(c2 - c1, 1.0)
    binw = roi / _OUT
    kidx = jax.lax.broadcasted_iota(jnp.int32, (1, 16), 1)
    k = kidx.astype(jnp.float32)
    pos = c1 + (k + 0.5) * 0.5 * binw
    valid = (pos > -1.0) & (pos < dim) & (kidx < 14)
    x = jnp.maximum(pos, 0.0)
    xl0 = jnp.floor(x).astype(jnp.int32)
    cond = xl0 >= dim - 1
    xl = jnp.where(cond, dim - 1, xl0)
    xh = jnp.where(cond, dim - 1, xl0 + 1)
    xv = jnp.where(cond, xl.astype(jnp.float32), x)
    lx = xv - xl.astype(jnp.float32)
    hx = 1.0 - lx
    vf = valid.astype(jnp.float32)
    hx = hx * vf
    lx = lx * vf
    cols = jax.lax.broadcasted_iota(jnp.int32, (b, 16, dim), 2)
    r = (jnp.where(cols == xl[:, :, None], hx[:, :, None], 0.0)
         + jnp.where(cols == xh[:, :, None], lx[:, :, None], 0.0))
    return 0.5 * r.reshape(b, 8, 2, dim).sum(axis=2)


def _roi_kernel(props_ref, f2_ref, out_ref):
    p = props_ref[...]
    x1 = p[:, 0:1] * _SCALE
    y1 = p[:, 1:2] * _SCALE
    x2 = p[:, 2:3] * _SCALE
    y2 = p[:, 3:4] * _SCALE
    ay = _interp_matrix(y1, y2, _H).reshape(_B * 8, _H)
    ax = _interp_matrix(x1, x2, _W).reshape(_B * 8, _W)
    ayb = jnp.broadcast_to(ay[None], (_C, _B * 8, _H))
    axb = jnp.broadcast_to(ax[None], (_C, _B * 8, _W))

    f3 = f2_ref[...]
    t = jax.lax.dot_general(
        ayb, f3,
        (((2,), (1,)), ((0,), (0,))),
        preferred_element_type=jnp.float32)
    o2 = jax.lax.dot_general(
        t, axb,
        (((2,), (2,)), ((0,), (0,))),
        preferred_element_type=jnp.float32)
    for b in range(_B):
        blk = o2[:, 8 * b:8 * b + _OUT, 8 * b:8 * b + _OUT]
        out_ref[b] = blk


@jax.jit
def kernel(feat, props):
    propst = props.T
    grid = _N // _B
    out = pl.pallas_call(
        _roi_kernel,
        grid=(grid,),
        in_specs=[
            pl.BlockSpec((_B, 4), lambda i: (i, 0)),
            pl.BlockSpec((_C, _H, _W), lambda i: (0, 0, 0)),
        ],
        out_specs=pl.BlockSpec((_B, _C, _OUT, _OUT), lambda i: (i, 0, 0, 0)),
        out_shape=jax.ShapeDtypeStruct((_N, _C, _OUT, _OUT), jnp.float32),
    )(propst, feat[0])
    return out


# speedup vs baseline: 7.8123x; 1.2723x over previous
import functools

import jax
import jax.numpy as jnp
from jax.experimental import pallas as pl

_C, _H, _W = 96, 64, 64
_OUT = 7
_SCALE = 1.0 / 16.0
_N = 1000
_B = 8


def _interp_matrix(c1, c2, dim):
    b = c1.shape[0]
    roi = jnp.maximum(c2 - c1, 1.0)
    binw = roi / _OUT
    kidx = jax.lax.broadcasted_iota(jnp.int32, (1, 16), 1)
    k = kidx.astype(jnp.float32)
    pos = c1 + (k + 0.5) * 0.5 * binw
    valid = (pos > -1.0) & (pos < dim) & (kidx < 14)
    x = jnp.maximum(pos, 0.0)
    xl0 = jnp.floor(x).astype(jnp.int32)
    cond = xl0 >= dim - 1
    xl = jnp.where(cond, dim - 1, xl0)
    xh = jnp.where(cond, dim - 1, xl0 + 1)
    xv = jnp.where(cond, xl.astype(jnp.float32), x)
    lx = xv - xl.astype(jnp.float32)
    hx = 1.0 - lx
    vf = valid.astype(jnp.float32)
    hx = hx * vf
    lx = lx * vf
    cols = jax.lax.broadcasted_iota(jnp.int32, (b, 16, dim), 2)
    r = (jnp.where(cols == xl[:, :, None], hx[:, :, None], 0.0)
         + jnp.where(cols == xh[:, :, None], lx[:, :, None], 0.0))
    return 0.5 * r.reshape(b, 8, 2, dim).sum(axis=2)


def _roi_kernel(props_ref, f2_ref, out_ref):
    p = props_ref[...]
    x1 = p[:, 0:1] * _SCALE
    y1 = p[:, 1:2] * _SCALE
    x2 = p[:, 2:3] * _SCALE
    y2 = p[:, 3:4] * _SCALE
    ay = _interp_matrix(y1, y2, _H).reshape(_B * 8, _H)
    ax = _interp_matrix(x1, x2, _W).reshape(_B * 8, _W)
    ayb = jnp.broadcast_to(ay[None], (_C, _B * 8, _H))
    axb = jnp.broadcast_to(ax[None], (_C, _B * 8, _W))

    f3 = f2_ref[...]
    t = jax.lax.dot_general(
        ayb, f3,
        (((2,), (1,)), ((0,), (0,))),
        preferred_element_type=jnp.float32)
    o2 = jax.lax.dot_general(
        t, axb,
        (((2,), (2,)), ((0,), (0,))),
        preferred_element_type=jnp.float32)
    for b in range(_B):
        cols = o2[:, :, 8 * b:8 * b + _OUT]
        out_ref[b] = jnp.concatenate(
            [cols[:, 8 * b + p, :] for p in range(_OUT)], axis=1)


@jax.jit
def kernel(feat, props):
    propst = props.T
    grid = _N // _B
    out = pl.pallas_call(
        _roi_kernel,
        grid=(grid,),
        in_specs=[
            pl.BlockSpec((_B, 4), lambda i: (i, 0)),
            pl.BlockSpec((_C, _H, _W), lambda i: (0, 0, 0)),
        ],
        out_specs=pl.BlockSpec((_B, _C, _OUT * _OUT), lambda i: (i, 0, 0)),
        out_shape=jax.ShapeDtypeStruct((_N, _C, _OUT * _OUT), jnp.float32),
    )(propst, feat[0])
    return out.reshape(_N, _C, _OUT, _OUT)


# speedup vs baseline: 7.8173x; 1.0006x over previous
import functools

import jax
import jax.numpy as jnp
from jax.experimental import pallas as pl

_C, _H, _W = 96, 64, 64
_OUT = 7
_SCALE = 1.0 / 16.0
_N = 1000
_B = 8


def _interp_matrix(c1, c2, dim):
    b = c1.shape[0]
    roi = jnp.maximum(c2 - c1, 1.0)
    binw = roi / _OUT
    kidx = jax.lax.broadcasted_iota(jnp.int32, (1, 16), 1)
    k = kidx.astype(jnp.float32)
    pos = c1 + (k + 0.5) * 0.5 * binw
    valid = (pos > -1.0) & (pos < dim) & (kidx < 14)
    x = jnp.maximum(pos, 0.0)
    xl0 = jnp.floor(x).astype(jnp.int32)
    cond = xl0 >= dim - 1
    xl = jnp.where(cond, dim - 1, xl0)
    xh = jnp.where(cond, dim - 1, xl0 + 1)
    xv = jnp.where(cond, xl.astype(jnp.float32), x)
    lx = xv - xl.astype(jnp.float32)
    hx = 1.0 - lx
    vf = valid.astype(jnp.float32)
    hx = hx * vf
    lx = lx * vf
    cols = jax.lax.broadcasted_iota(jnp.int32, (b, 16, dim), 2)
    r = (jnp.where(cols == xl[:, :, None], hx[:, :, None], 0.0)
         + jnp.where(cols == xh[:, :, None], lx[:, :, None], 0.0))
    return 0.5 * r.reshape(b, 8, 2, dim).sum(axis=2)


def _roi_kernel(props_ref, f2_ref, out_ref):
    p = props_ref[...]
    x1 = p[:, 0:1] * _SCALE
    y1 = p[:, 1:2] * _SCALE
    x2 = p[:, 2:3] * _SCALE
    y2 = p[:, 3:4] * _SCALE
    ay = _interp_matrix(y1, y2, _H).reshape(_B * 8, _H)
    ax = _interp_matrix(x1, x2, _W).reshape(_B * 8, _W)
    ayb = jnp.broadcast_to(ay[None], (_C, _B * 8, _H))
    axb = jnp.broadcast_to(ax[None], (_C, _B * 8, _W))

    f3 = f2_ref[...]
    t = jax.lax.dot_general(
        ayb, f3,
        (((2,), (1,)), ((0,), (0,))),
        preferred_element_type=jnp.float32)
    o2 = jax.lax.dot_general(
        t, axb,
        (((2,), (2,)), ((0,), (0,))),
        preferred_element_type=jnp.float32)
    for b in range(_B):
        blk = o2[:, 8 * b:8 * b + 8, 8 * b:8 * b + 8]
        out_ref[b] = jnp.concatenate(
            [blk[:, p, 0:_OUT] for p in range(_OUT)], axis=1)


@jax.jit
def kernel(feat, props):
    propst = props.T
    grid = _N // _B
    out = pl.pallas_call(
        _roi_kernel,
        grid=(grid,),
        in_specs=[
            pl.BlockSpec((_B, 4), lambda i: (i, 0)),
            pl.BlockSpec((_C, _H, _W), lambda i: (0, 0, 0)),
        ],
        out_specs=pl.BlockSpec((_B, _C, _OUT * _OUT), lambda i: (i, 0, 0)),
        out_shape=jax.ShapeDtypeStruct((_N, _C, _OUT * _OUT), jnp.float32),
    )(propst, feat[0])
    return out.reshape(_N, _C, _OUT, _OUT)


# speedup vs baseline: 9.4931x; 1.2144x over previous
import functools

import jax
import jax.numpy as jnp
from jax.experimental import pallas as pl

_C, _H, _W = 96, 64, 64
_OUT = 7
_SCALE = 1.0 / 16.0
_N = 1000
_B = 8


def _interp_matrix(c1, c2, dim):
    b = c1.shape[0]
    roi = jnp.maximum(c2 - c1, 1.0)
    binw = roi / _OUT
    kidx = jax.lax.broadcasted_iota(jnp.int32, (1, 16), 1)
    k = kidx.astype(jnp.float32)
    pos = c1 + (k + 0.5) * 0.5 * binw
    valid = (pos > -1.0) & (pos < dim) & (kidx < 14)
    x = jnp.maximum(pos, 0.0)
    xl0 = jnp.floor(x).astype(jnp.int32)
    cond = xl0 >= dim - 1
    xl = jnp.where(cond, dim - 1, xl0)
    xh = jnp.where(cond, dim - 1, xl0 + 1)
    xv = jnp.where(cond, xl.astype(jnp.float32), x)
    lx = xv - xl.astype(jnp.float32)
    hx = 1.0 - lx
    vf = valid.astype(jnp.float32)
    hx = hx * vf
    lx = lx * vf
    cols = jax.lax.broadcasted_iota(jnp.int32, (b, 16, dim), 2)
    r = (jnp.where(cols == xl[:, :, None], hx[:, :, None], 0.0)
         + jnp.where(cols == xh[:, :, None], lx[:, :, None], 0.0))
    return 0.5 * r.reshape(b, 8, 2, dim).sum(axis=2)


def _roi_kernel(props_ref, f2_ref, out_ref):
    p = props_ref[...]
    x1 = p[:, 0:1] * _SCALE
    y1 = p[:, 1:2] * _SCALE
    x2 = p[:, 2:3] * _SCALE
    y2 = p[:, 3:4] * _SCALE
    ay = _interp_matrix(y1, y2, _H).reshape(_B * 8, _H)
    ax = _interp_matrix(x1, x2, _W).reshape(_B * 8, _W)
    ayb = jnp.broadcast_to(ay[None], (_C, _B * 8, _H))

    f3 = f2_ref[...]
    t = jax.lax.dot_general(
        ayb, f3,
        (((2,), (1,)), ((0,), (0,))),
        preferred_element_type=jnp.float32)

    axt = ax.T
    pr = jax.lax.broadcasted_iota(jnp.int32, (8, 49), 0)
    jc = jax.lax.broadcasted_iota(jnp.int32, (8, 49), 1)
    msk = ((jc >= _OUT * pr) & (jc < _OUT * pr + _OUT)).astype(jnp.float32)
    for b in range(_B):
        tb = t[:, 8 * b:8 * b + 8, :]
        axtb = axt[:, 8 * b:8 * b + _OUT]
        rhsb = jnp.concatenate([axtb] * _OUT, axis=1)
        o3 = jax.lax.dot_general(
            tb, rhsb, (((2,), (0,)), ((), ())),
            preferred_element_type=jnp.float32)
        out_ref[b] = jnp.sum(o3 * msk[None], axis=1)


@jax.jit
def kernel(feat, props):
    propst = props.T
    grid = _N // _B
    out = pl.pallas_call(
        _roi_kernel,
        grid=(grid,),
        in_specs=[
            pl.BlockSpec((_B, 4), lambda i: (i, 0)),
            pl.BlockSpec((_C, _H, _W), lambda i: (0, 0, 0)),
        ],
        out_specs=pl.BlockSpec((_B, _C, _OUT * _OUT), lambda i: (i, 0, 0)),
        out_shape=jax.ShapeDtypeStruct((_N, _C, _OUT * _OUT), jnp.float32),
    )(propst, feat[0])
    return out.reshape(_N, _C, _OUT, _OUT)
